# Initial kernel scaffold; baseline (speedup 1.0000x reference)
import jax
import jax.numpy as jnp
from jax.experimental import pallas as pl


def kernel(x, embed_table, pos_table):
    raise NotImplementedError("write your pallas kernel here")


# speedup vs baseline: 7.7998x; 7.7998x over previous
import functools
import math

import jax
import jax.numpy as jnp
from jax import lax
from jax.experimental import pallas as pl
from jax.experimental.pallas import tpu as pltpu
from jax.experimental.pallas import tpu_sc as plsc

INPUT_DIM = 100000
EMBED_DIM = 128
SEQ = 60
BATCH = 4096

NC = 2
NS = 16
NW = NC * NS
ROWS_PER_W = BATCH // NW
RPS = 2
STEPS = ROWS_PER_W // RPS
IDX_PER_STEP = RPS * SEQ
NLANE = 16
NVEC = EMBED_DIM // NLANE
SCALE = 1.0 / (math.sqrt(513.0) * math.sqrt(60.0))

_mesh = plsc.VectorSubcoreMesh(
    core_axis_name="c", subcore_axis_name="s", num_cores=NC, num_subcores=NS
)


@functools.partial(
    pl.kernel,
    out_type=jax.ShapeDtypeStruct((BATCH, EMBED_DIM), jnp.float32),
    mesh=_mesh,
    scratch_types=[
        pltpu.VMEM((STEPS, IDX_PER_STEP), jnp.int32),
        pltpu.VMEM((IDX_PER_STEP, EMBED_DIM), jnp.float32),
        pltpu.VMEM((ROWS_PER_W, EMBED_DIM), jnp.float32),
        pltpu.VMEM((SEQ, EMBED_DIM), jnp.float32),
        pltpu.SemaphoreType.DMA,
    ],
)
def _encoder(x_hbm, tab_hbm, pos_hbm, out_hbm, idx_v, buf_v, out_v, pos_v, sem):
    wid = lax.axis_index("s") * NC + lax.axis_index("c")

    pltpu.sync_copy(x_hbm.at[wid], idx_v)
    pltpu.sync_copy(pos_hbm, pos_v)

    zeros8 = tuple(jnp.zeros((NLANE,), jnp.float32) for _ in range(NVEC))

    def pos_body(s, acc):
        return tuple(
            acc[k] + pos_v[s, pl.ds(NLANE * k, NLANE)] for k in range(NVEC)
        )

    psum = lax.fori_loop(0, SEQ, pos_body, zeros8)

    def step(j, carry):
        cp = pltpu.async_copy(tab_hbm.at[idx_v.at[j]], buf_v, sem)
        cp.wait()

        def s_body(s, acc):
            a0 = tuple(
                acc[0][k] + buf_v[s, pl.ds(NLANE * k, NLANE)]
                for k in range(NVEC)
            )
            a1 = tuple(
                acc[1][k] + buf_v[SEQ + s, pl.ds(NLANE * k, NLANE)]
                for k in range(NVEC)
            )
            return (a0, a1)

        acc = lax.fori_loop(0, SEQ, s_body, (zeros8, zeros8))
        for r in range(RPS):
            for k in range(NVEC):
                out_v[RPS * j + r, pl.ds(NLANE * k, NLANE)] = (
                    acc[r][k] + psum[k]
                ) * SCALE
        return carry

    lax.fori_loop(0, STEPS, step, 0)
    pltpu.sync_copy(out_v, out_hbm.at[pl.ds(wid * ROWS_PER_W, ROWS_PER_W)])


def kernel(x, embed_table, pos_table):
    x2 = x.astype(jnp.int32).reshape(NW, STEPS, IDX_PER_STEP)
    return _encoder(x2, embed_table, pos_table)


# speedup vs baseline: 16.3487x; 2.0960x over previous
import functools
import math

import jax
import jax.numpy as jnp
from jax import lax
from jax.experimental import pallas as pl
from jax.experimental.pallas import tpu as pltpu
from jax.experimental.pallas import tpu_sc as plsc

INPUT_DIM = 100000
EMBED_DIM = 128
SEQ = 60
BATCH = 4096

NC = 2
NS = 16
NW = NC * NS
ROWS_PER_W = BATCH // NW
NLANE = 16
NVEC = EMBED_DIM // NLANE
SCALE = 1.0 / (math.sqrt(513.0) * math.sqrt(60.0))

_mesh = plsc.VectorSubcoreMesh(
    core_axis_name="c", subcore_axis_name="s", num_cores=NC, num_subcores=NS
)


@functools.partial(
    pl.kernel,
    out_type=jax.ShapeDtypeStruct((BATCH, EMBED_DIM), jnp.float32),
    mesh=_mesh,
    scratch_types=[
        pltpu.VMEM((SEQ, ROWS_PER_W), jnp.int32),
        pltpu.VMEM((ROWS_PER_W, EMBED_DIM), jnp.float32),
        pltpu.VMEM((SEQ, EMBED_DIM), jnp.float32),
        pltpu.SemaphoreType.DMA,
    ],
)
def _encoder(xw_hbm, tab_hbm, pos_hbm, out_hbm, idx_v, acc_v, pos_v, sem):
    wid = lax.axis_index("s") * NC + lax.axis_index("c")

    pltpu.sync_copy(xw_hbm.at[wid], idx_v)
    pltpu.sync_copy(pos_hbm, pos_v)

    pltpu.async_copy(tab_hbm.at[idx_v.at[0]], acc_v, sem).wait()

    def fire(s, c):
        pltpu.async_copy(tab_hbm.at[idx_v.at[s]], acc_v, sem, add=True)
        return c

    lax.fori_loop(1, SEQ, fire, 0)

    def drain(s, c):
        pltpu.make_async_copy(
            tab_hbm.at[pl.ds(0, ROWS_PER_W)], acc_v, sem
        ).wait()
        return c

    lax.fori_loop(1, SEQ, drain, 0)

    zeros8 = tuple(jnp.zeros((NLANE,), jnp.float32) for _ in range(NVEC))

    def pos_body(s, acc):
        return tuple(
            acc[k] + pos_v[s, pl.ds(NLANE * k, NLANE)] for k in range(NVEC)
        )

    psum = lax.fori_loop(0, SEQ, pos_body, zeros8)

    def fin(r, c):
        for k in range(NVEC):
            sl = pl.ds(NLANE * k, NLANE)
            acc_v[r, sl] = (acc_v[r, sl] + psum[k]) * SCALE
        return c

    lax.fori_loop(0, ROWS_PER_W, fin, 0)
    pltpu.sync_copy(acc_v, out_hbm.at[pl.ds(wid * ROWS_PER_W, ROWS_PER_W)])


def kernel(x, embed_table, pos_table):
    xw = (
        x.astype(jnp.int32)
        .reshape(NW, ROWS_PER_W, SEQ)
        .transpose(0, 2, 1)
    )
    return _encoder(xw, embed_table, pos_table)


# speedup vs baseline: 16.6726x; 1.0198x over previous
import functools
import math

import jax
import jax.numpy as jnp
from jax import lax
from jax.experimental import pallas as pl
from jax.experimental.pallas import tpu as pltpu
from jax.experimental.pallas import tpu_sc as plsc

INPUT_DIM = 100000
EMBED_DIM = 128
SEQ = 60
BATCH = 4096

NC = 2
NS = 16
NW = NC * NS
ROWS_PER_W = BATCH // NW
NLANE = 16
NVEC = EMBED_DIM // NLANE
SCALE = 1.0 / (math.sqrt(513.0) * math.sqrt(60.0))

_mesh = plsc.VectorSubcoreMesh(
    core_axis_name="c", subcore_axis_name="s", num_cores=NC, num_subcores=NS
)


@functools.partial(
    pl.kernel,
    out_type=jax.ShapeDtypeStruct((BATCH, EMBED_DIM), jnp.float32),
    mesh=_mesh,
    scratch_types=[
        pltpu.VMEM((SEQ, ROWS_PER_W), jnp.int32),
        pltpu.VMEM((ROWS_PER_W, EMBED_DIM), jnp.float32),
        pltpu.VMEM((SEQ, EMBED_DIM), jnp.float32),
        pltpu.SemaphoreType.DMA,
        pltpu.SemaphoreType.DMA,
    ],
)
def _encoder(
    xw_hbm, tab_hbm, pos_hbm, out_hbm, idx_v, acc_v, pos_v, sem, pos_sem
):
    wid = lax.axis_index("s") * NC + lax.axis_index("c")

    pos_cp = pltpu.async_copy(pos_hbm, pos_v, pos_sem)
    pltpu.sync_copy(xw_hbm.at[wid], idx_v)

    init_cp = pltpu.async_copy(tab_hbm.at[idx_v.at[0]], acc_v, sem)

    pos_cp.wait()
    zeros8 = tuple(jnp.zeros((NLANE,), jnp.float32) for _ in range(NVEC))

    def pos_body(s, acc):
        return tuple(
            acc[k] + pos_v[s, pl.ds(NLANE * k, NLANE)] for k in range(NVEC)
        )

    psum = lax.fori_loop(0, SEQ, pos_body, zeros8)
    init_cp.wait()

    def fire(s, c):
        pltpu.async_copy(tab_hbm.at[idx_v.at[s]], acc_v, sem, add=True)
        return c

    lax.fori_loop(1, SEQ, fire, 0)

    def drain(s, c):
        pltpu.make_async_copy(
            tab_hbm.at[pl.ds(0, ROWS_PER_W)], acc_v, sem
        ).wait()
        return c

    lax.fori_loop(1, SEQ, drain, 0)

    def fin(r, c):
        for k in range(NVEC):
            sl = pl.ds(NLANE * k, NLANE)
            acc_v[r, sl] = (acc_v[r, sl] + psum[k]) * SCALE
        return c

    lax.fori_loop(0, ROWS_PER_W, fin, 0)
    pltpu.sync_copy(acc_v, out_hbm.at[pl.ds(wid * ROWS_PER_W, ROWS_PER_W)])


def kernel(x, embed_table, pos_table):
    xw = (
        x.astype(jnp.int32)
        .reshape(NW, ROWS_PER_W, SEQ)
        .transpose(0, 2, 1)
    )
    return _encoder(xw, embed_table, pos_table)


# speedup vs baseline: 16.6784x; 1.0003x over previous
import functools
import math

import jax
import jax.numpy as jnp
from jax import lax
from jax.experimental import pallas as pl
from jax.experimental.pallas import tpu as pltpu
from jax.experimental.pallas import tpu_sc as plsc

INPUT_DIM = 100000
EMBED_DIM = 128
SEQ = 60
BATCH = 4096

NC = 2
NS = 16
NW = NC * NS
ROWS_PER_W = BATCH // NW
HALF = ROWS_PER_W // 2
NLANE = 16
NVEC = EMBED_DIM // NLANE
SCALE = 1.0 / (math.sqrt(513.0) * math.sqrt(60.0))

_mesh = plsc.VectorSubcoreMesh(
    core_axis_name="c", subcore_axis_name="s", num_cores=NC, num_subcores=NS
)


@functools.partial(
    pl.kernel,
    out_type=jax.ShapeDtypeStruct((BATCH, EMBED_DIM), jnp.float32),
    mesh=_mesh,
    scratch_types=[
        pltpu.VMEM((SEQ, ROWS_PER_W), jnp.int32),
        pltpu.VMEM((ROWS_PER_W, EMBED_DIM), jnp.float32),
        pltpu.VMEM((SEQ, EMBED_DIM), jnp.float32),
        pltpu.SemaphoreType.DMA,
        pltpu.SemaphoreType.DMA,
        pltpu.SemaphoreType.DMA,
    ],
)
def _encoder(
    xw_hbm, tab_hbm, pos_hbm, out_hbm, idx_v, acc_v, pos_v, sem_a, sem_b, sem_x
):
    wid = lax.axis_index("s") * NC + lax.axis_index("c")
    out_base = wid * ROWS_PER_W

    pos_cp = pltpu.async_copy(pos_hbm, pos_v, sem_x)
    pltpu.sync_copy(xw_hbm.at[wid], idx_v)

    def idx_at(s, h):
        return idx_v.at[s, pl.ds(h * HALF, HALF)]

    def acc_at(h):
        return acc_v.at[pl.ds(h * HALF, HALF)]

    init_a = pltpu.async_copy(tab_hbm.at[idx_at(0, 0)], acc_at(0), sem_a)
    init_b = pltpu.async_copy(tab_hbm.at[idx_at(0, 1)], acc_at(1), sem_b)

    pos_cp.wait()
    zeros8 = tuple(jnp.zeros((NLANE,), jnp.float32) for _ in range(NVEC))

    def pos_body(s, acc):
        return tuple(
            acc[k] + pos_v[s, pl.ds(NLANE * k, NLANE)] for k in range(NVEC)
        )

    psum = lax.fori_loop(0, SEQ, pos_body, zeros8)
    init_a.wait()
    init_b.wait()

    def fire(s, c):
        pltpu.async_copy(tab_hbm.at[idx_at(s, 0)], acc_at(0), sem_a, add=True)
        pltpu.async_copy(tab_hbm.at[idx_at(s, 1)], acc_at(1), sem_b, add=True)
        return c

    lax.fori_loop(1, SEQ, fire, 0)

    def drain(sem, h):
        def body(s, c):
            pltpu.make_async_copy(
                tab_hbm.at[pl.ds(0, HALF)], acc_at(h), sem
            ).wait()
            return c

        lax.fori_loop(1, SEQ, body, 0)

    def fin(h):
        def body(r, c):
            row = h * HALF + r
            for k in range(NVEC):
                sl = pl.ds(NLANE * k, NLANE)
                acc_v[row, sl] = (acc_v[row, sl] + psum[k]) * SCALE
            return c

        lax.fori_loop(0, HALF, body, 0)

    drain(sem_a, 0)
    fin(0)
    out_a = pltpu.async_copy(
        acc_at(0), out_hbm.at[pl.ds(out_base, HALF)], sem_x
    )
    drain(sem_b, 1)
    fin(1)
    pltpu.sync_copy(acc_at(1), out_hbm.at[pl.ds(out_base + HALF, HALF)])
    out_a.wait()


def kernel(x, embed_table, pos_table):
    xw = (
        x.astype(jnp.int32)
        .reshape(NW, ROWS_PER_W, SEQ)
        .transpose(0, 2, 1)
    )
    return _encoder(xw, embed_table, pos_table)


# speedup vs baseline: 16.7016x; 1.0014x over previous
import functools
import math

import jax
import jax.numpy as jnp
from jax import lax
from jax.experimental import pallas as pl
from jax.experimental.pallas import tpu as pltpu
from jax.experimental.pallas import tpu_sc as plsc

INPUT_DIM = 100000
EMBED_DIM = 128
SEQ = 60
BATCH = 4096

NC = 2
NS = 16
NW = NC * NS
ROWS_PER_W = BATCH // NW
HALF = ROWS_PER_W // 2
NLANE = 16
NVEC = EMBED_DIM // NLANE
SCALE = 1.0 / (math.sqrt(513.0) * math.sqrt(60.0))

_mesh = plsc.VectorSubcoreMesh(
    core_axis_name="c", subcore_axis_name="s", num_cores=NC, num_subcores=NS
)


@functools.partial(
    pl.kernel,
    out_type=jax.ShapeDtypeStruct((BATCH, EMBED_DIM), jnp.float32),
    mesh=_mesh,
    scratch_types=[
        pltpu.VMEM((SEQ, ROWS_PER_W), jnp.int32),
        pltpu.VMEM((ROWS_PER_W, EMBED_DIM), jnp.float32),
        pltpu.VMEM((SEQ, EMBED_DIM), jnp.float32),
        pltpu.SemaphoreType.DMA,
        pltpu.SemaphoreType.DMA,
        pltpu.SemaphoreType.DMA,
    ],
)
def _encoder(
    xw_hbm, tab_hbm, pos_hbm, out_hbm, idx_v, acc_v, pos_v, sem_a, sem_b, sem_x
):
    wid = lax.axis_index("s") * NC + lax.axis_index("c")
    out_base = wid * ROWS_PER_W

    pos_cp = pltpu.async_copy(pos_hbm, pos_v, sem_x)
    pltpu.sync_copy(xw_hbm.at[wid], idx_v)

    def idx_at(s, h):
        return idx_v.at[s, pl.ds(h * HALF, HALF)]

    def acc_at(h):
        return acc_v.at[pl.ds(h * HALF, HALF)]

    init_a = pltpu.async_copy(tab_hbm.at[idx_at(0, 0)], acc_at(0), sem_a)
    init_b = pltpu.async_copy(tab_hbm.at[idx_at(0, 1)], acc_at(1), sem_b)

    pos_cp.wait()
    zeros8 = tuple(jnp.zeros((NLANE,), jnp.float32) for _ in range(NVEC))

    def pos_body(s, acc):
        return tuple(
            acc[k] + pos_v[s, pl.ds(NLANE * k, NLANE)] for k in range(NVEC)
        )

    psum = lax.fori_loop(0, SEQ, pos_body, zeros8)
    init_a.wait()
    init_b.wait()

    def fire(s, c):
        pltpu.async_copy(tab_hbm.at[idx_at(s, 0)], acc_at(0), sem_a, add=True)
        pltpu.async_copy(tab_hbm.at[idx_at(s, 1)], acc_at(1), sem_b, add=True)
        return c

    lax.fori_loop(1, SEQ, fire, 0)

    def drain(sem, h):
        def body(s, c):
            pltpu.make_async_copy(
                tab_hbm.at[pl.ds(0, HALF)], acc_at(h), sem
            ).wait()
            return c

        lax.fori_loop(1, SEQ, body, 0)

    def fin(h):
        def body(r, c):
            row = h * HALF + 2 * r
            for dr in range(2):
                for k in range(NVEC):
                    sl = pl.ds(NLANE * k, NLANE)
                    acc_v[row + dr, sl] = (acc_v[row + dr, sl] + psum[k]) * SCALE
            return c

        lax.fori_loop(0, HALF // 2, body, 0)

    drain(sem_a, 0)
    fin(0)
    out_a = pltpu.async_copy(
        acc_at(0), out_hbm.at[pl.ds(out_base, HALF)], sem_x
    )
    drain(sem_b, 1)
    fin(1)
    pltpu.sync_copy(acc_at(1), out_hbm.at[pl.ds(out_base + HALF, HALF)])
    out_a.wait()


def kernel(x, embed_table, pos_table):
    xw = (
        x.astype(jnp.int32)
        .reshape(NW, ROWS_PER_W, SEQ)
        .transpose(0, 2, 1)
    )
    return _encoder(xw, embed_table, pos_table)
